# baseline (speedup 1.0000x reference)
import functools

import jax
import jax.numpy as jnp
from jax import lax
from jax.experimental import pallas as pl
from jax.experimental.pallas import tpu as pltpu
from jax.experimental.pallas import tpu_sc as plsc

_K = 8192
_C = 32
_N = 8192
_TN = 128
_TK = 256
_NKT = _K // _TK
_COMMITMENT = 0.25

_NC = 2
_NS = 16
_NW = _NC * _NS
_BPW = _N // _NW
_CHUNK = 128
_NCHUNK = _BPW // _CHUNK


_LANES = 128


def _argmin_body(x_ref, w_ref, idx_ref, dsum_ref, csum_ref):
    i = pl.program_id(0)

    @pl.when(i == 0)
    def _precompute():
        csum_ref[...] = lax.dot_general(
            jnp.ones((1, _C), jnp.float32), w_ref[...] * w_ref[...],
            (((1,), (1,)), ((), ())), preferred_element_type=jnp.float32)
        dsum_ref[...] = jnp.zeros_like(dsum_ref)

    x = jnp.transpose(x_ref[0], (1, 0))
    a = jnp.sum(x * x, axis=1, keepdims=True)
    xs = x * 2.0

    bestv = jnp.full((_TN, _LANES), jnp.inf, jnp.float32)
    bidxv = jnp.zeros((_TN, _LANES), jnp.int32)
    lane = lax.broadcasted_iota(jnp.int32, (_TN, _LANES), 1)

    for j in range(_NKT):
        w = w_ref[j * _TK:(j + 1) * _TK, :]
        xw2 = lax.dot_general(xs, w, (((1,), (1,)), ((), ())),
                              preferred_element_type=jnp.float32)
        for p in range(_TK // _LANES):
            base = j * _TK + p * _LANES
            dch = (a - xw2[:, p * _LANES:(p + 1) * _LANES]) \
                + csum_ref[:, base:base + _LANES]
            take = dch < bestv
            bestv = jnp.where(take, dch, bestv)
            bidxv = jnp.where(take, lane + base, bidxv)

    m = jnp.min(bestv, axis=1, keepdims=True)
    cand = jnp.where(bestv == m, bidxv, jnp.int32(2**30))
    idx_ref[...] = jnp.min(cand, axis=1, keepdims=True)
    dsum_ref[...] += jnp.sum(m, keepdims=True)


def _sc_gather_body(w_hbm, idx_hbm, out_hbm, idx_v, rows_v, sem):
    wid = lax.axis_index("s") * _NC + lax.axis_index("c")
    base = wid * _BPW
    pltpu.sync_copy(idx_hbm.at[wid], idx_v)
    copies = [
        pltpu.async_copy(w_hbm.at[idx_v.at[j]], rows_v.at[j], sem)
        for j in range(_NCHUNK)
    ]
    for cp in copies:
        cp.wait()
    for j in range(_NCHUNK):
        pltpu.sync_copy(rows_v.at[j], out_hbm.at[pl.ds(base + j * _CHUNK, _CHUNK)])


_sc_gather = functools.partial(
    pl.kernel,
    out_type=jax.ShapeDtypeStruct((_N, _C), jnp.float32),
    mesh=plsc.VectorSubcoreMesh(core_axis_name="c", subcore_axis_name="s"),
    scratch_types=[
        pltpu.VMEM((_NCHUNK, _CHUNK), jnp.int32),
        pltpu.VMEM((_NCHUNK, _CHUNK, _C), jnp.float32),
        pltpu.SemaphoreType.DMA,
    ],
    compiler_params=pltpu.CompilerParams(use_tc_tiling_on_sc=False),
)(_sc_gather_body)


def kernel(inputs, weights):
    b, c, h, w = inputs.shape
    x_cm = inputs.reshape(b, c, h * w)
    tok_per_b = h * w
    seg_per_b = tok_per_b // _TN

    idx, dsum = pl.pallas_call(
        _argmin_body,
        grid=(_N // _TN,),
        in_specs=[
            pl.BlockSpec((1, _C, _TN),
                         lambda i: (i // seg_per_b, 0, i % seg_per_b)),
            pl.BlockSpec((_K, _C), lambda i: (0, 0)),
        ],
        out_specs=[
            pl.BlockSpec((_TN, 1), lambda i: (i, 0)),
            pl.BlockSpec((1, 1), lambda i: (0, 0)),
        ],
        out_shape=[
            jax.ShapeDtypeStruct((_N, 1), jnp.int32),
            jax.ShapeDtypeStruct((1, 1), jnp.float32),
        ],
        scratch_shapes=[pltpu.VMEM((1, _K), jnp.float32)],
    )(x_cm, weights)

    idx3 = idx.reshape(_NW, _NCHUNK, _CHUNK)
    out_flat = _sc_gather(weights, idx3)

    m = dsum[0, 0] / jnp.float32(_N * _C)
    loss = m + _COMMITMENT * m
    quantized = out_flat.reshape(b, h, w, c)
    quantized = jnp.transpose(quantized, (0, 3, 1, 2))
    return (quantized, loss)


# speedup vs baseline: 1.1716x; 1.1716x over previous
import functools

import jax
import jax.numpy as jnp
from jax import lax
from jax.experimental import pallas as pl
from jax.experimental.pallas import tpu as pltpu
from jax.experimental.pallas import tpu_sc as plsc

_K = 8192
_C = 32
_N = 8192
_TN = 256
_TK = 256
_NKT = _K // _TK
_COMMITMENT = 0.25

_NC = 2
_NS = 16
_NW = _NC * _NS
_BPW = _N // _NW
_CHUNK = 128
_NCHUNK = _BPW // _CHUNK


_LANES = 128


def _argmin_body(x_ref, w_ref, idx_ref, dsum_ref, csum_ref):
    i = pl.program_id(0)

    @pl.when(i == 0)
    def _precompute():
        csum_ref[...] = lax.dot_general(
            jnp.ones((1, _C), jnp.float32), w_ref[...] * w_ref[...],
            (((1,), (1,)), ((), ())), preferred_element_type=jnp.float32)
        dsum_ref[...] = jnp.zeros_like(dsum_ref)

    x = x_ref[...]
    a = jnp.sum(x * x, axis=1, keepdims=True)
    xs = x * 2.0

    bestv = jnp.full((_TN, _LANES), jnp.inf, jnp.float32)
    bidxv = jnp.zeros((_TN, _LANES), jnp.int32)
    lane = lax.broadcasted_iota(jnp.int32, (_TN, _LANES), 1)

    for j in range(_NKT):
        w = w_ref[j * _TK:(j + 1) * _TK, :]
        xw2 = lax.dot_general(xs, w, (((1,), (1,)), ((), ())),
                              preferred_element_type=jnp.float32)
        for p in range(_TK // _LANES):
            base = j * _TK + p * _LANES
            dch = (a - xw2[:, p * _LANES:(p + 1) * _LANES]) \
                + csum_ref[:, base:base + _LANES]
            take = dch < bestv
            bestv = jnp.where(take, dch, bestv)
            bidxv = jnp.where(take, lane + base, bidxv)

    m = jnp.min(bestv, axis=1, keepdims=True)
    cand = jnp.where(bestv == m, bidxv, jnp.int32(2**30))
    idx_ref[...] = jnp.min(cand, axis=1, keepdims=True)
    dsum_ref[...] += jnp.sum(m, keepdims=True)


def _sc_gather_body(w_hbm, idx_hbm, out_hbm, idx_v, rows_v, sem):
    wid = lax.axis_index("s") * _NC + lax.axis_index("c")
    base = wid * _BPW
    pltpu.sync_copy(idx_hbm.at[wid], idx_v)
    copies = [
        pltpu.async_copy(w_hbm.at[idx_v.at[j]], rows_v.at[j], sem)
        for j in range(_NCHUNK)
    ]
    for cp in copies:
        cp.wait()
    for j in range(_NCHUNK):
        pltpu.sync_copy(rows_v.at[j], out_hbm.at[pl.ds(base + j * _CHUNK, _CHUNK)])


_sc_gather = functools.partial(
    pl.kernel,
    out_type=jax.ShapeDtypeStruct((_N, _C), jnp.float32),
    mesh=plsc.VectorSubcoreMesh(core_axis_name="c", subcore_axis_name="s"),
    scratch_types=[
        pltpu.VMEM((_NCHUNK, _CHUNK), jnp.int32),
        pltpu.VMEM((_NCHUNK, _CHUNK, _C), jnp.float32),
        pltpu.SemaphoreType.DMA,
    ],
    compiler_params=pltpu.CompilerParams(use_tc_tiling_on_sc=False),
)(_sc_gather_body)


def kernel(inputs, weights):
    b, c, h, w = inputs.shape
    flatten = jnp.transpose(inputs, (0, 2, 3, 1))
    flat = flatten.reshape(-1, _C)

    idx, dsum = pl.pallas_call(
        _argmin_body,
        grid=(_N // _TN,),
        in_specs=[
            pl.BlockSpec((_TN, _C), lambda i: (i, 0)),
            pl.BlockSpec((_K, _C), lambda i: (0, 0)),
        ],
        out_specs=[
            pl.BlockSpec((_TN, 1), lambda i: (i, 0)),
            pl.BlockSpec((1, 1), lambda i: (0, 0)),
        ],
        out_shape=[
            jax.ShapeDtypeStruct((_N, 1), jnp.int32),
            jax.ShapeDtypeStruct((1, 1), jnp.float32),
        ],
        scratch_shapes=[pltpu.VMEM((1, _K), jnp.float32)],
    )(flat, weights)

    idx3 = idx.reshape(_NW, _NCHUNK, _CHUNK)
    out_flat = _sc_gather(weights, idx3)

    m = dsum[0, 0] / jnp.float32(_N * _C)
    loss = m + _COMMITMENT * m
    quantized = out_flat.reshape(b, h, w, c)
    quantized = jnp.transpose(quantized, (0, 3, 1, 2))
    return (quantized, loss)


# speedup vs baseline: 1.2022x; 1.0262x over previous
import functools

import jax
import jax.numpy as jnp
from jax import lax
from jax.experimental import pallas as pl
from jax.experimental.pallas import tpu as pltpu
from jax.experimental.pallas import tpu_sc as plsc

_K = 8192
_C = 32
_N = 8192
_TN = 512
_TK = 256
_NKT = _K // _TK
_COMMITMENT = 0.25

_NC = 2
_NS = 16
_NW = _NC * _NS
_BPW = _N // _NW
_CHUNK = 128
_NCHUNK = _BPW // _CHUNK


_LANES = 128


def _argmin_body(x_ref, w_ref, idx_ref, dsum_ref, csum_ref):
    i = pl.program_id(0)

    @pl.when(i == 0)
    def _precompute():
        csum_ref[...] = lax.dot_general(
            jnp.ones((1, _C), jnp.float32), w_ref[...] * w_ref[...],
            (((1,), (1,)), ((), ())), preferred_element_type=jnp.float32)
        dsum_ref[...] = jnp.zeros_like(dsum_ref)

    x = x_ref[...]
    a = jnp.sum(x * x, axis=1, keepdims=True)
    xs = x * 2.0

    bestv = jnp.full((_TN, _LANES), jnp.inf, jnp.float32)
    bidxv = jnp.zeros((_TN, _LANES), jnp.int32)
    lane = lax.broadcasted_iota(jnp.int32, (_TN, _LANES), 1)

    for j in range(_NKT):
        w = w_ref[j * _TK:(j + 1) * _TK, :]
        xw2 = lax.dot_general(xs, w, (((1,), (1,)), ((), ())),
                              preferred_element_type=jnp.float32)
        for p in range(_TK // _LANES):
            base = j * _TK + p * _LANES
            dch = (a - xw2[:, p * _LANES:(p + 1) * _LANES]) \
                + csum_ref[:, base:base + _LANES]
            take = dch < bestv
            bestv = jnp.where(take, dch, bestv)
            bidxv = jnp.where(take, lane + base, bidxv)

    m = jnp.min(bestv, axis=1, keepdims=True)
    cand = jnp.where(bestv == m, bidxv, jnp.int32(2**30))
    idx_ref[...] = jnp.min(cand, axis=1, keepdims=True)
    dsum_ref[...] += jnp.sum(m, keepdims=True)


def _sc_gather_body(w_hbm, idx_hbm, out_hbm, idx_v, rows_v, sem):
    wid = lax.axis_index("s") * _NC + lax.axis_index("c")
    base = wid * _BPW
    pltpu.sync_copy(idx_hbm.at[wid], idx_v)
    copies = [
        pltpu.async_copy(w_hbm.at[idx_v.at[j]], rows_v.at[j], sem)
        for j in range(_NCHUNK)
    ]
    for cp in copies:
        cp.wait()
    for j in range(_NCHUNK):
        pltpu.sync_copy(rows_v.at[j], out_hbm.at[pl.ds(base + j * _CHUNK, _CHUNK)])


_sc_gather = functools.partial(
    pl.kernel,
    out_type=jax.ShapeDtypeStruct((_N, _C), jnp.float32),
    mesh=plsc.VectorSubcoreMesh(core_axis_name="c", subcore_axis_name="s"),
    scratch_types=[
        pltpu.VMEM((_NCHUNK, _CHUNK), jnp.int32),
        pltpu.VMEM((_NCHUNK, _CHUNK, _C), jnp.float32),
        pltpu.SemaphoreType.DMA,
    ],
    compiler_params=pltpu.CompilerParams(use_tc_tiling_on_sc=False),
)(_sc_gather_body)


def kernel(inputs, weights):
    b, c, h, w = inputs.shape
    flatten = jnp.transpose(inputs, (0, 2, 3, 1))
    flat = flatten.reshape(-1, _C)

    idx, dsum = pl.pallas_call(
        _argmin_body,
        grid=(_N // _TN,),
        in_specs=[
            pl.BlockSpec((_TN, _C), lambda i: (i, 0)),
            pl.BlockSpec((_K, _C), lambda i: (0, 0)),
        ],
        out_specs=[
            pl.BlockSpec((_TN, 1), lambda i: (i, 0)),
            pl.BlockSpec((1, 1), lambda i: (0, 0)),
        ],
        out_shape=[
            jax.ShapeDtypeStruct((_N, 1), jnp.int32),
            jax.ShapeDtypeStruct((1, 1), jnp.float32),
        ],
        scratch_shapes=[pltpu.VMEM((1, _K), jnp.float32)],
    )(flat, weights)

    idx3 = idx.reshape(_NW, _NCHUNK, _CHUNK)
    out_flat = _sc_gather(weights, idx3)

    m = dsum[0, 0] / jnp.float32(_N * _C)
    loss = m + _COMMITMENT * m
    quantized = out_flat.reshape(b, h, w, c)
    quantized = jnp.transpose(quantized, (0, 3, 1, 2))
    return (quantized, loss)


# speedup vs baseline: 1.2310x; 1.0240x over previous
import functools

import jax
import jax.numpy as jnp
from jax import lax
from jax.experimental import pallas as pl
from jax.experimental.pallas import tpu as pltpu
from jax.experimental.pallas import tpu_sc as plsc

_K = 8192
_C = 32
_N = 8192
_TN = 1024
_TK = 256
_NKT = _K // _TK
_COMMITMENT = 0.25

_NC = 2
_NS = 16
_NW = _NC * _NS
_BPW = _N // _NW
_CHUNK = 128
_NCHUNK = _BPW // _CHUNK


_LANES = 128


def _argmin_body(x_ref, w_ref, idx_ref, dsum_ref, csum_ref):
    i = pl.program_id(0)

    @pl.when(i == 0)
    def _precompute():
        csum_ref[...] = lax.dot_general(
            jnp.ones((1, _C), jnp.float32), w_ref[...] * w_ref[...],
            (((1,), (1,)), ((), ())), preferred_element_type=jnp.float32)
        dsum_ref[...] = jnp.zeros_like(dsum_ref)

    x = x_ref[...]
    a = jnp.sum(x * x, axis=1, keepdims=True)
    xs = x * 2.0

    bestv = jnp.full((_TN, _LANES), jnp.inf, jnp.float32)
    bidxv = jnp.zeros((_TN, _LANES), jnp.int32)
    lane = lax.broadcasted_iota(jnp.int32, (_TN, _LANES), 1)

    for j in range(_NKT):
        w = w_ref[j * _TK:(j + 1) * _TK, :]
        xw2 = lax.dot_general(xs, w, (((1,), (1,)), ((), ())),
                              preferred_element_type=jnp.float32)
        for p in range(_TK // _LANES):
            base = j * _TK + p * _LANES
            dch = (a - xw2[:, p * _LANES:(p + 1) * _LANES]) \
                + csum_ref[:, base:base + _LANES]
            take = dch < bestv
            bestv = jnp.where(take, dch, bestv)
            bidxv = jnp.where(take, lane + base, bidxv)

    m = jnp.min(bestv, axis=1, keepdims=True)
    cand = jnp.where(bestv == m, bidxv, jnp.int32(2**30))
    idx_ref[...] = jnp.min(cand, axis=1, keepdims=True)
    dsum_ref[...] += jnp.sum(m, keepdims=True)


def _sc_gather_body(w_hbm, idx_hbm, out_hbm, idx_v, rows_v, sem):
    wid = lax.axis_index("s") * _NC + lax.axis_index("c")
    base = wid * _BPW
    pltpu.sync_copy(idx_hbm.at[wid], idx_v)
    copies = [
        pltpu.async_copy(w_hbm.at[idx_v.at[j]], rows_v.at[j], sem)
        for j in range(_NCHUNK)
    ]
    for cp in copies:
        cp.wait()
    for j in range(_NCHUNK):
        pltpu.sync_copy(rows_v.at[j], out_hbm.at[pl.ds(base + j * _CHUNK, _CHUNK)])


_sc_gather = functools.partial(
    pl.kernel,
    out_type=jax.ShapeDtypeStruct((_N, _C), jnp.float32),
    mesh=plsc.VectorSubcoreMesh(core_axis_name="c", subcore_axis_name="s"),
    scratch_types=[
        pltpu.VMEM((_NCHUNK, _CHUNK), jnp.int32),
        pltpu.VMEM((_NCHUNK, _CHUNK, _C), jnp.float32),
        pltpu.SemaphoreType.DMA,
    ],
    compiler_params=pltpu.CompilerParams(use_tc_tiling_on_sc=False),
)(_sc_gather_body)


def kernel(inputs, weights):
    b, c, h, w = inputs.shape
    flatten = jnp.transpose(inputs, (0, 2, 3, 1))
    flat = flatten.reshape(-1, _C)

    idx, dsum = pl.pallas_call(
        _argmin_body,
        grid=(_N // _TN,),
        in_specs=[
            pl.BlockSpec((_TN, _C), lambda i: (i, 0)),
            pl.BlockSpec((_K, _C), lambda i: (0, 0)),
        ],
        out_specs=[
            pl.BlockSpec((_TN, 1), lambda i: (i, 0)),
            pl.BlockSpec((1, 1), lambda i: (0, 0)),
        ],
        out_shape=[
            jax.ShapeDtypeStruct((_N, 1), jnp.int32),
            jax.ShapeDtypeStruct((1, 1), jnp.float32),
        ],
        scratch_shapes=[pltpu.VMEM((1, _K), jnp.float32)],
    )(flat, weights)

    idx3 = idx.reshape(_NW, _NCHUNK, _CHUNK)
    out_flat = _sc_gather(weights, idx3)

    m = dsum[0, 0] / jnp.float32(_N * _C)
    loss = m + _COMMITMENT * m
    quantized = out_flat.reshape(b, h, w, c)
    quantized = jnp.transpose(quantized, (0, 3, 1, 2))
    return (quantized, loss)


# speedup vs baseline: 1.3019x; 1.0575x over previous
import functools

import jax
import jax.numpy as jnp
from jax import lax
from jax.experimental import pallas as pl
from jax.experimental.pallas import tpu as pltpu
from jax.experimental.pallas import tpu_sc as plsc

_K = 8192
_C = 32
_N = 8192
_TN = 1024
_TK = 256
_NKT = _K // _TK
_COMMITMENT = 0.25

_NC = 2
_NS = 16
_NW = _NC * _NS
_BPW = _N // _NW
_CHUNK = 128
_NCHUNK = _BPW // _CHUNK


_LANES = 128


def _argmin_body(x_ref, w_ref, idx_ref, dsum_ref, csum_ref):
    i = pl.program_id(0)

    @pl.when(i == 0)
    def _precompute():
        csum_ref[...] = lax.dot_general(
            jnp.ones((1, _C), jnp.float32), w_ref[...] * w_ref[...],
            (((1,), (1,)), ((), ())), preferred_element_type=jnp.float32)
        dsum_ref[...] = jnp.zeros_like(dsum_ref)

    x = jnp.transpose(x_ref[0], (1, 0))
    a = jnp.sum(x * x, axis=1, keepdims=True)
    xs = x * 2.0

    bestv = jnp.full((_TN, _LANES), jnp.inf, jnp.float32)
    bidxv = jnp.zeros((_TN, _LANES), jnp.int32)
    lane = lax.broadcasted_iota(jnp.int32, (_TN, _LANES), 1)

    for j in range(_NKT):
        w = w_ref[j * _TK:(j + 1) * _TK, :]
        xw2 = lax.dot_general(xs, w, (((1,), (1,)), ((), ())),
                              preferred_element_type=jnp.float32)
        for p in range(_TK // _LANES):
            base = j * _TK + p * _LANES
            dch = (a - xw2[:, p * _LANES:(p + 1) * _LANES]) \
                + csum_ref[:, base:base + _LANES]
            take = dch < bestv
            bestv = jnp.where(take, dch, bestv)
            bidxv = jnp.where(take, lane + base, bidxv)

    m = jnp.min(bestv, axis=1, keepdims=True)
    cand = jnp.where(bestv == m, bidxv, jnp.int32(2**30))
    idx = jnp.min(cand, axis=1, keepdims=True)
    idx_ref[...] = idx.reshape(_TN // _BPW, _NCHUNK, _CHUNK)
    dsum_ref[...] += jnp.sum(m, keepdims=True)


def _sc_gather_body(w_hbm, idx_hbm, out_hbm, idx_v, rows_v, sem):
    wid = lax.axis_index("s") * _NC + lax.axis_index("c")
    base = wid * _BPW
    pltpu.sync_copy(idx_hbm.at[wid], idx_v)
    copies = [
        pltpu.async_copy(w_hbm.at[idx_v.at[j]], rows_v.at[j], sem)
        for j in range(_NCHUNK)
    ]
    for cp in copies:
        cp.wait()
    for j in range(_NCHUNK):
        pltpu.sync_copy(rows_v.at[j], out_hbm.at[pl.ds(base + j * _CHUNK, _CHUNK)])


_sc_gather = functools.partial(
    pl.kernel,
    out_type=jax.ShapeDtypeStruct((_N, _C), jnp.float32),
    mesh=plsc.VectorSubcoreMesh(core_axis_name="c", subcore_axis_name="s"),
    scratch_types=[
        pltpu.VMEM((_NCHUNK, _CHUNK), jnp.int32),
        pltpu.VMEM((_NCHUNK, _CHUNK, _C), jnp.float32),
        pltpu.SemaphoreType.DMA,
    ],
    compiler_params=pltpu.CompilerParams(use_tc_tiling_on_sc=False),
)(_sc_gather_body)


def kernel(inputs, weights):
    b, c, h, w = inputs.shape
    x_cm = inputs.reshape(b, c, h * w)

    idx3, dsum = pl.pallas_call(
        _argmin_body,
        grid=(_N // _TN,),
        in_specs=[
            pl.BlockSpec((1, _C, _TN), lambda i: (i, 0, 0)),
            pl.BlockSpec((_K, _C), lambda i: (0, 0)),
        ],
        out_specs=[
            pl.BlockSpec((_TN // _BPW, _NCHUNK, _CHUNK), lambda i: (i, 0, 0)),
            pl.BlockSpec((1, 1), lambda i: (0, 0)),
        ],
        out_shape=[
            jax.ShapeDtypeStruct((_NW, _NCHUNK, _CHUNK), jnp.int32),
            jax.ShapeDtypeStruct((1, 1), jnp.float32),
        ],
        scratch_shapes=[pltpu.VMEM((1, _K), jnp.float32)],
    )(x_cm, weights)

    out_flat = _sc_gather(weights, idx3)

    m = dsum[0, 0] / jnp.float32(_N * _C)
    loss = m + _COMMITMENT * m
    quantized = out_flat.reshape(b, h, w, c)
    quantized = jnp.transpose(quantized, (0, 3, 1, 2))
    return (quantized, loss)
